# baseline (speedup 1.0000x reference)
import functools

import jax
import jax.numpy as jnp
from jax import lax
from jax.experimental import pallas as pl
from jax.experimental.pallas import tpu as pltpu
from jax.experimental.pallas import tpu_sc as plsc

B, L = 4096, 50
V = 100000
D0, D1, D2 = 32, 16, 16
DP, D3 = 128, 32
DOUT = D0 + D1 + D2 + D3
N = B * L

NC, NS = 2, 16
NW = NC * NS
PER_W = N // NW
CHUNK = 256
NCHUNK = PER_W // CHUNK

NBUF = 2
NSTEP = NCHUNK // NBUF
TAIL = NCHUNK - NSTEP * NBUF

_OFFS = (0, D0, D0 + D1, D0 + D1 + D2)
_DIMS = (D0, D1, D2, D3)


_FROWS = 4000


def _fold_body(wd_ref, wr_ref, br_ref, out_ref):
    x = wd_ref[...].reshape(_FROWS // 4, 4, DP)
    wr = wr_ref[...]
    br = br_ref[...]
    for f in range(4):
        out_ref[:, pl.ds(f * D3, D3)] = (
            jnp.dot(x[:, f, :], wr, preferred_element_type=jnp.float32) + br)


def _fold_table(W_objectData, W_red, b_red):
    grid = (V // _FROWS,)
    folded = pl.pallas_call(
        _fold_body,
        grid=grid,
        in_specs=[
            pl.BlockSpec((_FROWS, DP), lambda i: (i, 0)),
            pl.BlockSpec((DP, D3), lambda i: (0, 0)),
            pl.BlockSpec((1, D3), lambda i: (0, 0)),
        ],
        out_specs=pl.BlockSpec((_FROWS * D3 // 128, 128), lambda i: (i, 0)),
        out_shape=jax.ShapeDtypeStruct((V * D3 // 128, 128), jnp.float32),
    )(W_objectData, W_red, b_red.reshape(1, D3))
    return folded.reshape(V, D3)


_RROWS = 8000


def _repack_body(s_ref, e_ref, o_ref, so_ref, eo_ref, oo_ref):
    for ref, oref, d in ((s_ref, so_ref, D0), (e_ref, eo_ref, D1),
                         (o_ref, oo_ref, D2)):
        g = 128 // d
        x = ref[...].reshape(_RROWS // g, g, d)
        for f in range(g):
            oref[:, pl.ds(f * d, d)] = x[:, f, :]


def _repack_tables(W_subject, W_event, W_objectType):
    grid = ((V + _RROWS - 1) // _RROWS,)
    outs = pl.pallas_call(
        _repack_body,
        grid=grid,
        in_specs=[
            pl.BlockSpec((_RROWS, D0), lambda i: (i, 0)),
            pl.BlockSpec((_RROWS, D1), lambda i: (i, 0)),
            pl.BlockSpec((_RROWS, D2), lambda i: (i, 0)),
        ],
        out_specs=[
            pl.BlockSpec((_RROWS * D0 // 128, 128), lambda i: (i, 0)),
            pl.BlockSpec((_RROWS * D1 // 128, 128), lambda i: (i, 0)),
            pl.BlockSpec((_RROWS * D2 // 128, 128), lambda i: (i, 0)),
        ],
        out_shape=[
            jax.ShapeDtypeStruct((V * D0 // 128, 128), jnp.float32),
            jax.ShapeDtypeStruct((V * D1 // 128, 128), jnp.float32),
            jax.ShapeDtypeStruct((V * D2 // 128, 128), jnp.float32),
        ],
    )(W_subject, W_event, W_objectType)
    return (outs[0].reshape(V, D0), outs[1].reshape(V, D1),
            outs[2].reshape(V, D2))


_H = CHUNK // 128


_ROWS_W = NCHUNK * 4 * _H


def _sc_body(idx_hbm, t0, t1, t2, t3, out_hbm,
             idx_v, bufs, gsems, wsems):
    wid = lax.axis_index("s") * NC + lax.axis_index("c")
    base = wid * PER_W
    tabs = (t0, t1, t2, t3)
    pltpu.sync_copy(idx_hbm.at[pl.ds(wid * _ROWS_W, _ROWS_W)], idx_v)

    def _gather_descs(b, c):
        ds = []
        for f in range(4):
            for h in range(_H):
                ds.append(pltpu.make_async_copy(
                    tabs[f].at[idx_v.at[(c * 4 + f) * _H + h]],
                    bufs[b][f].at[pl.ds(h * 128, 128)], gsems[b]))
        return ds

    def fire_gathers(b, c):
        for d in _gather_descs(b, c):
            d.start()

    def wait_gathers(b, c):
        for d in _gather_descs(b, c):
            d.wait()

    class _W:
        def __init__(self, b, c):
            rb = base + c * CHUNK
            self.ds = [
                pltpu.make_async_copy(
                    bufs[b][f],
                    out_hbm.at[pl.ds(rb, CHUNK), pl.ds(_OFFS[f], _DIMS[f])],
                    wsems[b])
                for f in range(4)
            ]

        def start(self):
            for d in self.ds:
                d.start()

        def wait(self):
            for d in self.ds:
                d.wait()

    write = _W

    for b in range(NBUF):
        fire_gathers(b, b)

    def body(s, _):
        c0 = s * NBUF
        for b in range(NBUF):
            wait_gathers(b, c0 + b)
            write(b, c0 + b).start()
        for b in range(NBUF):
            write(b, c0 + b).wait()
            fire_gathers(b, c0 + NBUF + b)
        return ()

    lax.fori_loop(0, NSTEP - 1, body, ())

    c0 = (NSTEP - 1) * NBUF
    for b in range(NBUF):
        wait_gathers(b, c0 + b)
        write(b, c0 + b).start()
    for b in range(NBUF):
        write(b, c0 + b).wait()
    for t in range(TAIL):
        c = NSTEP * NBUF + t
        b = t % NBUF
        fire_gathers(b, c)
        wait_gathers(b, c)
        write(b, c).start()
        write(b, c).wait()


_sc_gather = functools.partial(
    pl.kernel,
    out_type=jax.ShapeDtypeStruct((N, DOUT), jnp.float32),
    mesh=plsc.VectorSubcoreMesh(core_axis_name="c", subcore_axis_name="s"),
    compiler_params=pltpu.CompilerParams(use_tc_tiling_on_sc=False),
    scratch_types=[
        pltpu.VMEM((_ROWS_W, 128), jnp.int32),
        [[pltpu.VMEM((CHUNK, d), jnp.float32) for d in _DIMS]
         for _ in range(NBUF)],
        [pltpu.SemaphoreType.DMA for _ in range(NBUF)],
        [pltpu.SemaphoreType.DMA for _ in range(NBUF)],
    ],
)(_sc_body)


def kernel(input, W_subject, W_event, W_objectType, W_objectData, W_red, b_red):
    folded = _fold_table(W_objectData, W_red, b_red)
    W_subject, W_event, W_objectType = _repack_tables(
        W_subject, W_event, W_objectType)
    idx = (input.reshape(NW, NCHUNK, _H, 128, 4)
           .transpose(0, 1, 4, 2, 3)
           .reshape(NW * NCHUNK * 4 * _H, 128))
    out = _sc_gather(idx, W_subject, W_event, W_objectType, folded)
    return out.reshape(B, L, DOUT)


# speedup vs baseline: 1.1102x; 1.1102x over previous
import functools

import jax
import jax.numpy as jnp
from jax import lax
from jax.experimental import pallas as pl
from jax.experimental.pallas import tpu as pltpu
from jax.experimental.pallas import tpu_sc as plsc

B, L = 4096, 50
V = 100000
D0, D1, D2 = 32, 16, 16
DP, D3 = 128, 32
DOUT = D0 + D1 + D2 + D3
N = B * L

NC, NS = 2, 16
NW = NC * NS
PER_W = N // NW
CHUNK = 256
NCHUNK = PER_W // CHUNK

NBUF = 2
NSTEP = NCHUNK // NBUF
TAIL = NCHUNK - NSTEP * NBUF

_OFFS = (0, D0, D0 + D1, D0 + D1 + D2)
_DIMS = (D0, D1, D2, D3)


_FROWS = 4000


def _fold_body(wd_ref, wr_ref, br_ref, out_ref):
    x = wd_ref[...].reshape(_FROWS // 4, 4, DP)
    wr = wr_ref[...]
    br = br_ref[...]
    for f in range(4):
        out_ref[:, pl.ds(f * D3, D3)] = (
            jnp.dot(x[:, f, :], wr, preferred_element_type=jnp.float32) + br)


def _fold_table(W_objectData, W_red, b_red):
    grid = (V // _FROWS,)
    folded = pl.pallas_call(
        _fold_body,
        grid=grid,
        in_specs=[
            pl.BlockSpec((_FROWS, DP), lambda i: (i, 0)),
            pl.BlockSpec((DP, D3), lambda i: (0, 0)),
            pl.BlockSpec((1, D3), lambda i: (0, 0)),
        ],
        out_specs=pl.BlockSpec((_FROWS * D3 // 128, 128), lambda i: (i, 0)),
        out_shape=jax.ShapeDtypeStruct((V * D3 // 128, 128), jnp.float32),
    )(W_objectData, W_red, b_red.reshape(1, D3))
    return folded.reshape(V, D3)


_H = CHUNK // 128


_ROWS_W = NCHUNK * 4 * _H


def _sc_body(idx_hbm, t0, t1, t2, t3, out_hbm,
             idx_v, bufs, gsems, wsems):
    wid = lax.axis_index("s") * NC + lax.axis_index("c")
    base = wid * PER_W
    tabs = (t0, t1, t2, t3)
    pltpu.sync_copy(idx_hbm.at[pl.ds(wid * _ROWS_W, _ROWS_W)], idx_v)

    def _gather_descs(b, c):
        ds = []
        for f in range(4):
            for h in range(_H):
                ds.append(pltpu.make_async_copy(
                    tabs[f].at[idx_v.at[(c * 4 + f) * _H + h]],
                    bufs[b][f].at[pl.ds(h * 128, 128)], gsems[b]))
        return ds

    def fire_gathers(b, c):
        for d in _gather_descs(b, c):
            d.start()

    def wait_gathers(b, c):
        for d in _gather_descs(b, c):
            d.wait()

    class _W:
        def __init__(self, b, c):
            rb = base + c * CHUNK
            self.ds = [
                pltpu.make_async_copy(
                    bufs[b][f],
                    out_hbm.at[pl.ds(rb, CHUNK), pl.ds(_OFFS[f], _DIMS[f])],
                    wsems[b])
                for f in range(4)
            ]

        def start(self):
            for d in self.ds:
                d.start()

        def wait(self):
            for d in self.ds:
                d.wait()

    write = _W

    for b in range(NBUF):
        fire_gathers(b, b)

    def body(s, _):
        c0 = s * NBUF
        for b in range(NBUF):
            wait_gathers(b, c0 + b)
            write(b, c0 + b).start()
        for b in range(NBUF):
            write(b, c0 + b).wait()
            fire_gathers(b, c0 + NBUF + b)
        return ()

    lax.fori_loop(0, NSTEP - 1, body, ())

    c0 = (NSTEP - 1) * NBUF
    for b in range(NBUF):
        wait_gathers(b, c0 + b)
        write(b, c0 + b).start()
    for b in range(NBUF):
        write(b, c0 + b).wait()
    for t in range(TAIL):
        c = NSTEP * NBUF + t
        b = t % NBUF
        fire_gathers(b, c)
        wait_gathers(b, c)
        write(b, c).start()
        write(b, c).wait()


_sc_gather = functools.partial(
    pl.kernel,
    out_type=jax.ShapeDtypeStruct((N, DOUT), jnp.float32),
    mesh=plsc.VectorSubcoreMesh(core_axis_name="c", subcore_axis_name="s"),
    compiler_params=pltpu.CompilerParams(use_tc_tiling_on_sc=False),
    scratch_types=[
        pltpu.VMEM((_ROWS_W, 128), jnp.int32),
        [[pltpu.VMEM((CHUNK, d), jnp.float32) for d in _DIMS]
         for _ in range(NBUF)],
        [pltpu.SemaphoreType.DMA for _ in range(NBUF)],
        [pltpu.SemaphoreType.DMA for _ in range(NBUF)],
    ],
)(_sc_body)


def kernel(input, W_subject, W_event, W_objectType, W_objectData, W_red, b_red):
    folded = _fold_table(W_objectData, W_red, b_red)
    idx = (input.reshape(NW, NCHUNK, _H, 128, 4)
           .transpose(0, 1, 4, 2, 3)
           .reshape(NW * NCHUNK * 4 * _H, 128))
    out = _sc_gather(idx, W_subject, W_event, W_objectType, folded)
    return out.reshape(B, L, DOUT)
